# baseline (speedup 1.0000x reference)
import functools

import jax
import jax.numpy as jnp
from jax import lax
from jax.experimental import pallas as pl
from jax.experimental.pallas import tpu as pltpu
from jax.experimental.pallas import tpu_sc as plsc

NC = 2
NS = 16
NW = NC * NS
CH = 96
SECT = 40
NBUF = 2
F0 = 0.625
BR = 1000


def _cdiv(a, b):
    return (a + b - 1) // b


def _sc_aggregate(h, sd0, sd1, rows_pad):
    m_by_core = (sd0.shape[1], sd1.shape[1])
    F = h.shape[1]
    rows_per_tile = rows_pad // NS
    zfull = rows_per_tile // CH
    zrem = rows_per_tile - zfull * CH
    mesh = plsc.VectorSubcoreMesh(core_axis_name="c", subcore_axis_name="s")

    @functools.partial(
        pl.kernel,
        out_type=jax.ShapeDtypeStruct((NC, rows_pad, F), jnp.float32),
        mesh=mesh,
        scratch_types=[
            pltpu.VMEM((SECT, 2, CH), jnp.int32),
        ] + [pltpu.VMEM((CH, F), jnp.float32)] * NBUF
          + [pltpu.VMEM_SHARED((rows_pad, F), jnp.float32)]
          + [pltpu.SemaphoreType.DMA] * NBUF,
    )
    def agg_kernel(h_hbm, sd0_hbm, sd1_hbm, out_hbm, sdv, *rest):
        rows = rest[:NBUF]
        aggsh = rest[NBUF]
        sems = rest[NBUF + 1:]
        rowsa = rows[0]
        cid = lax.axis_index("c")
        sid = lax.axis_index("s")

        @pl.loop(0, CH)
        def _(r):
            @pl.loop(0, F, step=16)
            def _(c):
                rowsa[pl.ds(r, 1), pl.ds(c, 16)] = jnp.zeros((1, 16), jnp.float32)

        @pl.loop(0, zfull * CH, step=CH)
        def _(r):
            pltpu.sync_copy(rowsa, aggsh.at[pl.ds(sid * rows_per_tile + r, CH)])

        if zrem:
            pltpu.sync_copy(
                rowsa.at[pl.ds(0, zrem)],
                aggsh.at[pl.ds(sid * rows_per_tile + zfull * CH, zrem)],
            )

        plsc.subcore_barrier()

        def run_core(sd_hbm, m):
            for s0 in range(0, m, SECT):
                sw = min(SECT, m - s0)
                pltpu.sync_copy(sd_hbm.at[sid, pl.ds(s0, sw)], sdv.at[pl.ds(0, sw)])
                for k in range(NBUF - 1):
                    pltpu.async_copy(h_hbm.at[sdv.at[k, 0]], rows[k], sems[k])

                @pl.loop(0, sw, step=NBUF)
                def _(j):
                    for k in range(NBUF):
                        ki = (k + NBUF - 1) % NBUF
                        jn = jnp.where(j + k + NBUF - 1 < sw, j + k + NBUF - 1, 0)
                        pltpu.async_copy(h_hbm.at[sdv.at[jn, 0]], rows[ki], sems[ki])
                        pltpu.make_async_copy(h_hbm.at[sdv.at[0, 0]], rows[k], sems[k]).wait()
                        pltpu.sync_copy(rows[k], aggsh.at[sdv.at[j + k, 1]], add=True)

                for k in range(NBUF - 1):
                    pltpu.make_async_copy(h_hbm.at[sdv.at[0, 0]], rows[k], sems[k]).wait()

        @pl.when(cid == 0)
        def _():
            run_core(sd0_hbm, m_by_core[0])

        @pl.when(cid == 1)
        def _():
            run_core(sd1_hbm, m_by_core[1])

        plsc.subcore_barrier()
        pltpu.sync_copy(
            aggsh.at[pl.ds(sid * rows_per_tile, rows_per_tile)],
            out_hbm.at[cid, pl.ds(sid * rows_per_tile, rows_per_tile)],
        )

    return agg_kernel(h, sd0, sd1)


def _entry_kernel(x, W, b):
    n, f = x.shape

    def body(x_ref, w_ref, b_ref, o_ref):
        o_ref[...] = jnp.maximum(
            jnp.dot(x_ref[...], w_ref[...], preferred_element_type=jnp.float32)
            + b_ref[...], 0.0)

    return pl.pallas_call(
        body,
        grid=(n // BR,),
        in_specs=[
            pl.BlockSpec((BR, f), lambda i: (i, 0)),
            pl.BlockSpec(W.shape, lambda i: (0, 0)),
            pl.BlockSpec((1, f), lambda i: (0, 0)),
        ],
        out_specs=pl.BlockSpec((BR, W.shape[1]), lambda i: (i, 0)),
        out_shape=jax.ShapeDtypeStruct((n, W.shape[1]), jnp.float32),
    )(x, W, b)


def _mlp_kernel(h, parts, Wa, ba, Wb, bb):
    n, f = h.shape

    def body(h_ref, p_ref, wa_ref, ba_ref, wb_ref, bb_ref, o_ref):
        m = h_ref[...] + p_ref[0] + p_ref[1]
        t = jnp.maximum(
            jnp.dot(m, wa_ref[...], preferred_element_type=jnp.float32)
            + ba_ref[...], 0.0)
        o_ref[...] = jnp.maximum(
            jnp.dot(t, wb_ref[...], preferred_element_type=jnp.float32)
            + bb_ref[...], 0.0)

    return pl.pallas_call(
        body,
        grid=(n // BR,),
        in_specs=[
            pl.BlockSpec((BR, f), lambda i: (i, 0)),
            pl.BlockSpec((NC, BR, f), lambda i: (0, i, 0)),
            pl.BlockSpec((f, f), lambda i: (0, 0)),
            pl.BlockSpec((1, f), lambda i: (0, 0)),
            pl.BlockSpec((f, f), lambda i: (0, 0)),
            pl.BlockSpec((1, f), lambda i: (0, 0)),
        ],
        out_specs=pl.BlockSpec((BR, f), lambda i: (i, 0)),
        out_shape=jax.ShapeDtypeStruct((n, f), jnp.float32),
    )(h, parts, Wa, ba, Wb, bb)


def _final_kernel(h, parts, Wa, ba, Wb, bb, W1, b1, W2p, b2p, nclass):
    n, f = h.shape
    nblk = n // BR

    def body(h_ref, p_ref, wa_ref, ba_ref, wb_ref, bb_ref,
             w1_ref, b1_ref, w2_ref, b2_ref, o_ref, acc):
        i = pl.program_id(0)

        @pl.when(i == 0)
        def _():
            acc[...] = jnp.zeros_like(acc)

        m = h_ref[...] + p_ref[0] + p_ref[1]
        t = jnp.maximum(
            jnp.dot(m, wa_ref[...], preferred_element_type=jnp.float32)
            + ba_ref[...], 0.0)
        t = jnp.maximum(
            jnp.dot(t, wb_ref[...], preferred_element_type=jnp.float32)
            + bb_ref[...], 0.0)
        t = jnp.maximum(
            jnp.dot(t, w1_ref[...], preferred_element_type=jnp.float32)
            + b1_ref[...], 0.0)
        acc[...] += jnp.sum(t, axis=0, keepdims=True)

        @pl.when(i == nblk - 1)
        def _():
            pooled = acc[...] * (1.0 / n)
            logits = (jnp.dot(pooled, w2_ref[...],
                              preferred_element_type=jnp.float32) + b2_ref[...])
            lane = lax.broadcasted_iota(jnp.int32, (1, f), 1)
            valid = lane < nclass
            mx = jnp.max(jnp.where(valid, logits, -jnp.inf), axis=1, keepdims=True)
            e = jnp.where(valid, jnp.exp(logits - mx), 0.0)
            o_ref[...] = e / jnp.sum(e, axis=1, keepdims=True)

    return pl.pallas_call(
        body,
        grid=(nblk,),
        in_specs=[
            pl.BlockSpec((BR, f), lambda i: (i, 0)),
            pl.BlockSpec((NC, BR, f), lambda i: (0, i, 0)),
            pl.BlockSpec((f, f), lambda i: (0, 0)),
            pl.BlockSpec((1, f), lambda i: (0, 0)),
            pl.BlockSpec((f, f), lambda i: (0, 0)),
            pl.BlockSpec((1, f), lambda i: (0, 0)),
            pl.BlockSpec((f, f), lambda i: (0, 0)),
            pl.BlockSpec((1, f), lambda i: (0, 0)),
            pl.BlockSpec((f, f), lambda i: (0, 0)),
            pl.BlockSpec((1, f), lambda i: (0, 0)),
        ],
        out_specs=pl.BlockSpec((1, f), lambda i: (0, 0)),
        out_shape=jax.ShapeDtypeStruct((1, f), jnp.float32),
        scratch_shapes=[pltpu.VMEM((1, f), jnp.float32)],
    )(h, parts, Wa, ba, Wb, bb, W1, b1, W2p, b2p)


def kernel(x, edge_index, batch, params):
    n, f = x.shape
    e = edge_index.shape[1]
    nclass = params['W2'].shape[1]
    bn_eps = 1e-5

    def fold(W, b, g, be):
        s = g / jnp.sqrt(1.0 + bn_eps)
        return W * s[None, :], (b * s + be)[None, :]

    W0 = params['W0']
    b0 = params['b0'][None, :]
    convs = []
    for p in params['convs']:
        WaF, baF = fold(p['Wa'], p['ba'], p['g1'], p['be1'])
        WbF, bbF = fold(p['Wb'], p['bb'], p['g2'], p['be2'])
        convs.append((WaF, baF, WbF, bbF))
    W1 = params['W1']
    b1 = params['b1'][None, :]
    W2p = jnp.zeros((f, f), jnp.float32).at[:, :nclass].set(params['W2'])
    b2p = jnp.zeros((1, f), jnp.float32).at[:, :nclass].set(params['b2'])

    chunks_total = _cdiv(e, CH)
    m0 = max(NBUF, int(chunks_total * F0 / NS) // NBUF * NBUF)
    m1 = _cdiv(_cdiv(max(e - NS * m0 * CH, 0), CH), NS * NBUF) * NS * NBUF // NS
    e0 = NS * m0 * CH
    epad = NS * (m0 + m1) * CH
    rows_pad = _cdiv(n + 1, NS * 8) * NS * 8
    pad = epad - e
    src = edge_index[0]
    dst = edge_index[1]
    src_p = jnp.concatenate([src, jnp.zeros((pad,), src.dtype)])
    dst_p = jnp.concatenate([dst, jnp.full((pad,), n, dst.dtype)])
    sd0 = jnp.stack([src_p[:e0].reshape(NS, m0, CH),
                     dst_p[:e0].reshape(NS, m0, CH)], axis=2)
    sd1 = jnp.stack([src_p[e0:].reshape(NS, m1, CH),
                     dst_p[e0:].reshape(NS, m1, CH)], axis=2)

    h = _entry_kernel(x, W0, b0)
    for i in range(len(convs)):
        parts = _sc_aggregate(h, sd0, sd1, rows_pad)
        WaF, baF, WbF, bbF = convs[i]
        if i + 1 < len(convs):
            h = _mlp_kernel(h, parts, WaF, baF, WbF, bbF)
        else:
            out = _final_kernel(h, parts, WaF, baF, WbF, bbF, W1, b1, W2p, b2p, nclass)
    return out[:, :nclass]


# speedup vs baseline: 1.3320x; 1.3320x over previous
import functools

import jax
import jax.numpy as jnp
from jax import lax
from jax.experimental import pallas as pl
from jax.experimental.pallas import tpu as pltpu
from jax.experimental.pallas import tpu_sc as plsc

NC = 2
NS = 16
NW = NC * NS
CH = 72
SECT = 40
NBUF = 2
F0 = 0.625
BR = 1000


def _cdiv(a, b):
    return (a + b - 1) // b


def _sc_aggregate(h, sd0, sd1, rows_pad):
    m_by_core = (sd0.shape[1], sd1.shape[1])
    F = h.shape[1]
    rows_per_tile = rows_pad // NS
    zfull = rows_per_tile // CH
    zrem = rows_per_tile - zfull * CH
    mesh = plsc.VectorSubcoreMesh(core_axis_name="c", subcore_axis_name="s")

    @functools.partial(
        pl.kernel,
        out_type=jax.ShapeDtypeStruct((NC, rows_pad, F), jnp.float32),
        mesh=mesh,
        scratch_types=[
            pltpu.VMEM((SECT, 2, CH), jnp.int32),
        ] + [pltpu.VMEM((CH, F), jnp.float32)] * NBUF
          + [pltpu.VMEM_SHARED((rows_pad, F), jnp.float32)]
          + [pltpu.SemaphoreType.DMA] * NBUF,
    )
    def agg_kernel(h_hbm, sd0_hbm, sd1_hbm, out_hbm, sdv, *rest):
        rows = rest[:NBUF]
        aggsh = rest[NBUF]
        sems = rest[NBUF + 1:]
        rowsa = rows[0]
        cid = lax.axis_index("c")
        sid = lax.axis_index("s")

        @pl.loop(0, CH)
        def _(r):
            @pl.loop(0, F, step=16)
            def _(c):
                rowsa[pl.ds(r, 1), pl.ds(c, 16)] = jnp.zeros((1, 16), jnp.float32)

        @pl.loop(0, zfull * CH, step=CH)
        def _(r):
            pltpu.sync_copy(rowsa, aggsh.at[pl.ds(sid * rows_per_tile + r, CH)])

        if zrem:
            pltpu.sync_copy(
                rowsa.at[pl.ds(0, zrem)],
                aggsh.at[pl.ds(sid * rows_per_tile + zfull * CH, zrem)],
            )

        plsc.subcore_barrier()

        def run_core(sd_hbm, m):
            for s0 in range(0, m, SECT):
                sw = min(SECT, m - s0)
                pltpu.sync_copy(sd_hbm.at[sid, pl.ds(s0, sw)], sdv.at[pl.ds(0, sw)])
                for k in range(NBUF - 1):
                    pltpu.async_copy(h_hbm.at[sdv.at[k, 0]], rows[k], sems[k])

                @pl.loop(0, sw, step=NBUF)
                def _(j):
                    for k in range(NBUF):
                        ki = (k + NBUF - 1) % NBUF
                        jn = jnp.where(j + k + NBUF - 1 < sw, j + k + NBUF - 1, 0)
                        pltpu.async_copy(h_hbm.at[sdv.at[jn, 0]], rows[ki], sems[ki])
                        pltpu.make_async_copy(h_hbm.at[sdv.at[0, 0]], rows[k], sems[k]).wait()
                        pltpu.sync_copy(rows[k], aggsh.at[sdv.at[j + k, 1]], add=True)

                for k in range(NBUF - 1):
                    pltpu.make_async_copy(h_hbm.at[sdv.at[0, 0]], rows[k], sems[k]).wait()

        @pl.when(cid == 0)
        def _():
            run_core(sd0_hbm, m_by_core[0])

        @pl.when(cid == 1)
        def _():
            run_core(sd1_hbm, m_by_core[1])

        plsc.subcore_barrier()
        pltpu.sync_copy(
            aggsh.at[pl.ds(sid * rows_per_tile, rows_per_tile)],
            out_hbm.at[cid, pl.ds(sid * rows_per_tile, rows_per_tile)],
        )

    return agg_kernel(h, sd0, sd1)


def _entry_kernel(x, W, b):
    n, f = x.shape

    def body(x_ref, w_ref, b_ref, o_ref):
        o_ref[...] = jnp.maximum(
            jnp.dot(x_ref[...], w_ref[...], preferred_element_type=jnp.float32)
            + b_ref[...], 0.0)

    return pl.pallas_call(
        body,
        grid=(n // BR,),
        in_specs=[
            pl.BlockSpec((BR, f), lambda i: (i, 0)),
            pl.BlockSpec(W.shape, lambda i: (0, 0)),
            pl.BlockSpec((1, f), lambda i: (0, 0)),
        ],
        out_specs=pl.BlockSpec((BR, W.shape[1]), lambda i: (i, 0)),
        out_shape=jax.ShapeDtypeStruct((n, W.shape[1]), jnp.float32),
    )(x, W, b)


def _mlp_kernel(h, parts, Wa, ba, Wb, bb):
    n, f = h.shape

    def body(h_ref, p_ref, wa_ref, ba_ref, wb_ref, bb_ref, o_ref):
        m = h_ref[...] + p_ref[0] + p_ref[1]
        t = jnp.maximum(
            jnp.dot(m, wa_ref[...], preferred_element_type=jnp.float32)
            + ba_ref[...], 0.0)
        o_ref[...] = jnp.maximum(
            jnp.dot(t, wb_ref[...], preferred_element_type=jnp.float32)
            + bb_ref[...], 0.0)

    return pl.pallas_call(
        body,
        grid=(n // BR,),
        in_specs=[
            pl.BlockSpec((BR, f), lambda i: (i, 0)),
            pl.BlockSpec((NC, BR, f), lambda i: (0, i, 0)),
            pl.BlockSpec((f, f), lambda i: (0, 0)),
            pl.BlockSpec((1, f), lambda i: (0, 0)),
            pl.BlockSpec((f, f), lambda i: (0, 0)),
            pl.BlockSpec((1, f), lambda i: (0, 0)),
        ],
        out_specs=pl.BlockSpec((BR, f), lambda i: (i, 0)),
        out_shape=jax.ShapeDtypeStruct((n, f), jnp.float32),
    )(h, parts, Wa, ba, Wb, bb)


def _final_kernel(h, parts, Wa, ba, Wb, bb, W1, b1, W2p, b2p, nclass):
    n, f = h.shape
    nblk = n // BR

    def body(h_ref, p_ref, wa_ref, ba_ref, wb_ref, bb_ref,
             w1_ref, b1_ref, w2_ref, b2_ref, o_ref, acc):
        i = pl.program_id(0)

        @pl.when(i == 0)
        def _():
            acc[...] = jnp.zeros_like(acc)

        m = h_ref[...] + p_ref[0] + p_ref[1]
        t = jnp.maximum(
            jnp.dot(m, wa_ref[...], preferred_element_type=jnp.float32)
            + ba_ref[...], 0.0)
        t = jnp.maximum(
            jnp.dot(t, wb_ref[...], preferred_element_type=jnp.float32)
            + bb_ref[...], 0.0)
        t = jnp.maximum(
            jnp.dot(t, w1_ref[...], preferred_element_type=jnp.float32)
            + b1_ref[...], 0.0)
        acc[...] += jnp.sum(t, axis=0, keepdims=True)

        @pl.when(i == nblk - 1)
        def _():
            pooled = acc[...] * (1.0 / n)
            logits = (jnp.dot(pooled, w2_ref[...],
                              preferred_element_type=jnp.float32) + b2_ref[...])
            lane = lax.broadcasted_iota(jnp.int32, (1, f), 1)
            valid = lane < nclass
            mx = jnp.max(jnp.where(valid, logits, -jnp.inf), axis=1, keepdims=True)
            e = jnp.where(valid, jnp.exp(logits - mx), 0.0)
            o_ref[...] = e / jnp.sum(e, axis=1, keepdims=True)

    return pl.pallas_call(
        body,
        grid=(nblk,),
        in_specs=[
            pl.BlockSpec((BR, f), lambda i: (i, 0)),
            pl.BlockSpec((NC, BR, f), lambda i: (0, i, 0)),
            pl.BlockSpec((f, f), lambda i: (0, 0)),
            pl.BlockSpec((1, f), lambda i: (0, 0)),
            pl.BlockSpec((f, f), lambda i: (0, 0)),
            pl.BlockSpec((1, f), lambda i: (0, 0)),
            pl.BlockSpec((f, f), lambda i: (0, 0)),
            pl.BlockSpec((1, f), lambda i: (0, 0)),
            pl.BlockSpec((f, f), lambda i: (0, 0)),
            pl.BlockSpec((1, f), lambda i: (0, 0)),
        ],
        out_specs=pl.BlockSpec((1, f), lambda i: (0, 0)),
        out_shape=jax.ShapeDtypeStruct((1, f), jnp.float32),
        scratch_shapes=[pltpu.VMEM((1, f), jnp.float32)],
    )(h, parts, Wa, ba, Wb, bb, W1, b1, W2p, b2p)


def kernel(x, edge_index, batch, params):
    n, f = x.shape
    e = edge_index.shape[1]
    nclass = params['W2'].shape[1]
    bn_eps = 1e-5

    def fold(W, b, g, be):
        s = g / jnp.sqrt(1.0 + bn_eps)
        return W * s[None, :], (b * s + be)[None, :]

    W0 = params['W0']
    b0 = params['b0'][None, :]
    convs = []
    for p in params['convs']:
        WaF, baF = fold(p['Wa'], p['ba'], p['g1'], p['be1'])
        WbF, bbF = fold(p['Wb'], p['bb'], p['g2'], p['be2'])
        convs.append((WaF, baF, WbF, bbF))
    W1 = params['W1']
    b1 = params['b1'][None, :]
    W2p = jnp.zeros((f, f), jnp.float32).at[:, :nclass].set(params['W2'])
    b2p = jnp.zeros((1, f), jnp.float32).at[:, :nclass].set(params['b2'])

    chunks_total = _cdiv(e, CH)
    m0 = max(NBUF, int(chunks_total * F0 / NS) // NBUF * NBUF)
    m1 = _cdiv(_cdiv(max(e - NS * m0 * CH, 0), CH), NS * NBUF) * NS * NBUF // NS
    e0 = NS * m0 * CH
    epad = NS * (m0 + m1) * CH
    rows_pad = _cdiv(n + 1, NS * 8) * NS * 8
    pad = epad - e
    src = edge_index[0]
    dst = edge_index[1]
    src_p = jnp.concatenate([src, jnp.zeros((pad,), src.dtype)])
    dst_p = jnp.concatenate([dst, jnp.full((pad,), n, dst.dtype)])
    sd0 = jnp.stack([src_p[:e0].reshape(NS, m0, CH),
                     dst_p[:e0].reshape(NS, m0, CH)], axis=2)
    sd1 = jnp.stack([src_p[e0:].reshape(NS, m1, CH),
                     dst_p[e0:].reshape(NS, m1, CH)], axis=2)

    h = _entry_kernel(x, W0, b0)
    for i in range(len(convs)):
        parts = _sc_aggregate(h, sd0, sd1, rows_pad)
        WaF, baF, WbF, bbF = convs[i]
        if i + 1 < len(convs):
            h = _mlp_kernel(h, parts, WaF, baF, WbF, bbF)
        else:
            out = _final_kernel(h, parts, WaF, baF, WbF, bbF, W1, b1, W2p, b2p, nclass)
    return out[:, :nclass]


# speedup vs baseline: 1.3507x; 1.0140x over previous
import functools

import jax
import jax.numpy as jnp
from jax import lax
from jax.experimental import pallas as pl
from jax.experimental.pallas import tpu as pltpu
from jax.experimental.pallas import tpu_sc as plsc

NC = 2
NS = 16
NW = NC * NS
CH = 88
SECT = 40
NBUF = 2
F0 = 0.625
BR = 1000


def _cdiv(a, b):
    return (a + b - 1) // b


def _sc_aggregate(h, sd0, sd1, rows_pad):
    m_by_core = (sd0.shape[1], sd1.shape[1])
    F = h.shape[1]
    rows_per_tile = rows_pad // NS
    zfull = rows_per_tile // CH
    zrem = rows_per_tile - zfull * CH
    mesh = plsc.VectorSubcoreMesh(core_axis_name="c", subcore_axis_name="s")

    @functools.partial(
        pl.kernel,
        out_type=jax.ShapeDtypeStruct((NC, rows_pad, F), jnp.float32),
        mesh=mesh,
        scratch_types=[
            pltpu.VMEM((SECT, 2, CH), jnp.int32),
        ] + [pltpu.VMEM((CH, F), jnp.float32)] * NBUF
          + [pltpu.VMEM_SHARED((rows_pad, F), jnp.float32)]
          + [pltpu.SemaphoreType.DMA] * NBUF,
    )
    def agg_kernel(h_hbm, sd0_hbm, sd1_hbm, out_hbm, sdv, *rest):
        rows = rest[:NBUF]
        aggsh = rest[NBUF]
        sems = rest[NBUF + 1:]
        rowsa = rows[0]
        cid = lax.axis_index("c")
        sid = lax.axis_index("s")

        @pl.loop(0, CH)
        def _(r):
            @pl.loop(0, F, step=16)
            def _(c):
                rowsa[pl.ds(r, 1), pl.ds(c, 16)] = jnp.zeros((1, 16), jnp.float32)

        @pl.loop(0, zfull * CH, step=CH)
        def _(r):
            pltpu.sync_copy(rowsa, aggsh.at[pl.ds(sid * rows_per_tile + r, CH)])

        if zrem:
            pltpu.sync_copy(
                rowsa.at[pl.ds(0, zrem)],
                aggsh.at[pl.ds(sid * rows_per_tile + zfull * CH, zrem)],
            )

        plsc.subcore_barrier()

        def run_core(sd_hbm, m):
            for s0 in range(0, m, SECT):
                sw = min(SECT, m - s0)
                pltpu.sync_copy(sd_hbm.at[sid, pl.ds(s0, sw)], sdv.at[pl.ds(0, sw)])
                for k in range(NBUF - 1):
                    pltpu.async_copy(h_hbm.at[sdv.at[k, 0]], rows[k], sems[k])

                @pl.loop(0, sw, step=NBUF)
                def _(j):
                    for k in range(NBUF):
                        ki = (k + NBUF - 1) % NBUF
                        jn = jnp.where(j + k + NBUF - 1 < sw, j + k + NBUF - 1, 0)
                        pltpu.async_copy(h_hbm.at[sdv.at[jn, 0]], rows[ki], sems[ki])
                        pltpu.make_async_copy(h_hbm.at[sdv.at[0, 0]], rows[k], sems[k]).wait()
                        pltpu.sync_copy(rows[k], aggsh.at[sdv.at[j + k, 1]], add=True)

                for k in range(NBUF - 1):
                    pltpu.make_async_copy(h_hbm.at[sdv.at[0, 0]], rows[k], sems[k]).wait()

        @pl.when(cid == 0)
        def _():
            run_core(sd0_hbm, m_by_core[0])

        @pl.when(cid == 1)
        def _():
            run_core(sd1_hbm, m_by_core[1])

        plsc.subcore_barrier()
        pltpu.sync_copy(
            aggsh.at[pl.ds(sid * rows_per_tile, rows_per_tile)],
            out_hbm.at[cid, pl.ds(sid * rows_per_tile, rows_per_tile)],
        )

    return agg_kernel(h, sd0, sd1)


def _entry_kernel(x, W, b):
    n, f = x.shape

    def body(x_ref, w_ref, b_ref, o_ref):
        o_ref[...] = jnp.maximum(
            jnp.dot(x_ref[...], w_ref[...], preferred_element_type=jnp.float32)
            + b_ref[...], 0.0)

    return pl.pallas_call(
        body,
        grid=(n // BR,),
        in_specs=[
            pl.BlockSpec((BR, f), lambda i: (i, 0)),
            pl.BlockSpec(W.shape, lambda i: (0, 0)),
            pl.BlockSpec((1, f), lambda i: (0, 0)),
        ],
        out_specs=pl.BlockSpec((BR, W.shape[1]), lambda i: (i, 0)),
        out_shape=jax.ShapeDtypeStruct((n, W.shape[1]), jnp.float32),
    )(x, W, b)


def _mlp_kernel(h, parts, Wa, ba, Wb, bb):
    n, f = h.shape

    def body(h_ref, p_ref, wa_ref, ba_ref, wb_ref, bb_ref, o_ref):
        m = h_ref[...] + p_ref[0] + p_ref[1]
        t = jnp.maximum(
            jnp.dot(m, wa_ref[...], preferred_element_type=jnp.float32)
            + ba_ref[...], 0.0)
        o_ref[...] = jnp.maximum(
            jnp.dot(t, wb_ref[...], preferred_element_type=jnp.float32)
            + bb_ref[...], 0.0)

    return pl.pallas_call(
        body,
        grid=(n // BR,),
        in_specs=[
            pl.BlockSpec((BR, f), lambda i: (i, 0)),
            pl.BlockSpec((NC, BR, f), lambda i: (0, i, 0)),
            pl.BlockSpec((f, f), lambda i: (0, 0)),
            pl.BlockSpec((1, f), lambda i: (0, 0)),
            pl.BlockSpec((f, f), lambda i: (0, 0)),
            pl.BlockSpec((1, f), lambda i: (0, 0)),
        ],
        out_specs=pl.BlockSpec((BR, f), lambda i: (i, 0)),
        out_shape=jax.ShapeDtypeStruct((n, f), jnp.float32),
    )(h, parts, Wa, ba, Wb, bb)


def _final_kernel(h, parts, Wa, ba, Wb, bb, W1, b1, W2p, b2p, nclass):
    n, f = h.shape
    nblk = n // BR

    def body(h_ref, p_ref, wa_ref, ba_ref, wb_ref, bb_ref,
             w1_ref, b1_ref, w2_ref, b2_ref, o_ref, acc):
        i = pl.program_id(0)

        @pl.when(i == 0)
        def _():
            acc[...] = jnp.zeros_like(acc)

        m = h_ref[...] + p_ref[0] + p_ref[1]
        t = jnp.maximum(
            jnp.dot(m, wa_ref[...], preferred_element_type=jnp.float32)
            + ba_ref[...], 0.0)
        t = jnp.maximum(
            jnp.dot(t, wb_ref[...], preferred_element_type=jnp.float32)
            + bb_ref[...], 0.0)
        t = jnp.maximum(
            jnp.dot(t, w1_ref[...], preferred_element_type=jnp.float32)
            + b1_ref[...], 0.0)
        acc[...] += jnp.sum(t, axis=0, keepdims=True)

        @pl.when(i == nblk - 1)
        def _():
            pooled = acc[...] * (1.0 / n)
            logits = (jnp.dot(pooled, w2_ref[...],
                              preferred_element_type=jnp.float32) + b2_ref[...])
            lane = lax.broadcasted_iota(jnp.int32, (1, f), 1)
            valid = lane < nclass
            mx = jnp.max(jnp.where(valid, logits, -jnp.inf), axis=1, keepdims=True)
            e = jnp.where(valid, jnp.exp(logits - mx), 0.0)
            o_ref[...] = e / jnp.sum(e, axis=1, keepdims=True)

    return pl.pallas_call(
        body,
        grid=(nblk,),
        in_specs=[
            pl.BlockSpec((BR, f), lambda i: (i, 0)),
            pl.BlockSpec((NC, BR, f), lambda i: (0, i, 0)),
            pl.BlockSpec((f, f), lambda i: (0, 0)),
            pl.BlockSpec((1, f), lambda i: (0, 0)),
            pl.BlockSpec((f, f), lambda i: (0, 0)),
            pl.BlockSpec((1, f), lambda i: (0, 0)),
            pl.BlockSpec((f, f), lambda i: (0, 0)),
            pl.BlockSpec((1, f), lambda i: (0, 0)),
            pl.BlockSpec((f, f), lambda i: (0, 0)),
            pl.BlockSpec((1, f), lambda i: (0, 0)),
        ],
        out_specs=pl.BlockSpec((1, f), lambda i: (0, 0)),
        out_shape=jax.ShapeDtypeStruct((1, f), jnp.float32),
        scratch_shapes=[pltpu.VMEM((1, f), jnp.float32)],
    )(h, parts, Wa, ba, Wb, bb, W1, b1, W2p, b2p)


def kernel(x, edge_index, batch, params):
    n, f = x.shape
    e = edge_index.shape[1]
    nclass = params['W2'].shape[1]
    bn_eps = 1e-5

    def fold(W, b, g, be):
        s = g / jnp.sqrt(1.0 + bn_eps)
        return W * s[None, :], (b * s + be)[None, :]

    W0 = params['W0']
    b0 = params['b0'][None, :]
    convs = []
    for p in params['convs']:
        WaF, baF = fold(p['Wa'], p['ba'], p['g1'], p['be1'])
        WbF, bbF = fold(p['Wb'], p['bb'], p['g2'], p['be2'])
        convs.append((WaF, baF, WbF, bbF))
    W1 = params['W1']
    b1 = params['b1'][None, :]
    W2p = jnp.zeros((f, f), jnp.float32).at[:, :nclass].set(params['W2'])
    b2p = jnp.zeros((1, f), jnp.float32).at[:, :nclass].set(params['b2'])

    chunks_total = _cdiv(e, CH)
    m0 = max(NBUF, int(chunks_total * F0 / NS) // NBUF * NBUF)
    m1 = _cdiv(_cdiv(max(e - NS * m0 * CH, 0), CH), NS * NBUF) * NS * NBUF // NS
    e0 = NS * m0 * CH
    epad = NS * (m0 + m1) * CH
    rows_pad = _cdiv(n + 1, NS * 8) * NS * 8
    pad = epad - e
    src = edge_index[0]
    dst = edge_index[1]
    src_p = jnp.concatenate([src, jnp.zeros((pad,), src.dtype)])
    dst_p = jnp.concatenate([dst, jnp.full((pad,), n, dst.dtype)])
    sd0 = jnp.stack([src_p[:e0].reshape(NS, m0, CH),
                     dst_p[:e0].reshape(NS, m0, CH)], axis=2)
    sd1 = jnp.stack([src_p[e0:].reshape(NS, m1, CH),
                     dst_p[e0:].reshape(NS, m1, CH)], axis=2)

    h = _entry_kernel(x, W0, b0)
    for i in range(len(convs)):
        parts = _sc_aggregate(h, sd0, sd1, rows_pad)
        WaF, baF, WbF, bbF = convs[i]
        if i + 1 < len(convs):
            h = _mlp_kernel(h, parts, WaF, baF, WbF, bbF)
        else:
            out = _final_kernel(h, parts, WaF, baF, WbF, bbF, W1, b1, W2p, b2p, nclass)
    return out[:, :nclass]


# speedup vs baseline: 1.3985x; 1.0354x over previous
import functools

import jax
import jax.numpy as jnp
from jax import lax
from jax.experimental import pallas as pl
from jax.experimental.pallas import tpu as pltpu
from jax.experimental.pallas import tpu_sc as plsc

NC = 2
NS = 16
NW = NC * NS
CH = 80
SECT = 40
NBUF = 2
F0 = 0.625
BR = 1000


def _cdiv(a, b):
    return (a + b - 1) // b


def _sc_aggregate(h, sd0, sd1, rows_pad):
    m_by_core = (sd0.shape[1], sd1.shape[1])
    F = h.shape[1]
    rows_per_tile = rows_pad // NS
    zfull = rows_per_tile // CH
    zrem = rows_per_tile - zfull * CH
    mesh = plsc.VectorSubcoreMesh(core_axis_name="c", subcore_axis_name="s")

    @functools.partial(
        pl.kernel,
        out_type=jax.ShapeDtypeStruct((NC, rows_pad, F), jnp.float32),
        mesh=mesh,
        scratch_types=[
            pltpu.VMEM((SECT, 2, CH), jnp.int32),
        ] + [pltpu.VMEM((CH, F), jnp.float32)] * NBUF
          + [pltpu.VMEM_SHARED((rows_pad, F), jnp.float32)]
          + [pltpu.SemaphoreType.DMA] * NBUF,
    )
    def agg_kernel(h_hbm, sd0_hbm, sd1_hbm, out_hbm, sdv, *rest):
        rows = rest[:NBUF]
        aggsh = rest[NBUF]
        sems = rest[NBUF + 1:]
        rowsa = rows[0]
        cid = lax.axis_index("c")
        sid = lax.axis_index("s")

        @pl.loop(0, CH)
        def _(r):
            @pl.loop(0, F, step=16)
            def _(c):
                rowsa[pl.ds(r, 1), pl.ds(c, 16)] = jnp.zeros((1, 16), jnp.float32)

        @pl.loop(0, zfull * CH, step=CH)
        def _(r):
            pltpu.sync_copy(rowsa, aggsh.at[pl.ds(sid * rows_per_tile + r, CH)])

        if zrem:
            pltpu.sync_copy(
                rowsa.at[pl.ds(0, zrem)],
                aggsh.at[pl.ds(sid * rows_per_tile + zfull * CH, zrem)],
            )

        plsc.subcore_barrier()

        def run_core(sd_hbm, m):
            for s0 in range(0, m, SECT):
                sw = min(SECT, m - s0)
                pltpu.sync_copy(sd_hbm.at[sid, pl.ds(s0, sw)], sdv.at[pl.ds(0, sw)])
                for k in range(NBUF - 1):
                    pltpu.async_copy(h_hbm.at[sdv.at[k, 0]], rows[k], sems[k])

                @pl.loop(0, sw, step=NBUF)
                def _(j):
                    for k in range(NBUF):
                        ki = (k + NBUF - 1) % NBUF
                        jn = jnp.where(j + k + NBUF - 1 < sw, j + k + NBUF - 1, 0)
                        pltpu.async_copy(h_hbm.at[sdv.at[jn, 0]], rows[ki], sems[ki])
                        pltpu.make_async_copy(h_hbm.at[sdv.at[0, 0]], rows[k], sems[k]).wait()
                        pltpu.sync_copy(rows[k], aggsh.at[sdv.at[j + k, 1]], add=True)

                for k in range(NBUF - 1):
                    pltpu.make_async_copy(h_hbm.at[sdv.at[0, 0]], rows[k], sems[k]).wait()

        @pl.when(cid == 0)
        def _():
            run_core(sd0_hbm, m_by_core[0])

        @pl.when(cid == 1)
        def _():
            run_core(sd1_hbm, m_by_core[1])

        plsc.subcore_barrier()
        pltpu.sync_copy(
            aggsh.at[pl.ds(sid * rows_per_tile, rows_per_tile)],
            out_hbm.at[cid, pl.ds(sid * rows_per_tile, rows_per_tile)],
        )

    return agg_kernel(h, sd0, sd1)


def _entry_kernel(x, W, b):
    n, f = x.shape

    def body(x_ref, w_ref, b_ref, o_ref):
        o_ref[...] = jnp.maximum(
            jnp.dot(x_ref[...], w_ref[...], preferred_element_type=jnp.float32)
            + b_ref[...], 0.0)

    return pl.pallas_call(
        body,
        grid=(n // BR,),
        in_specs=[
            pl.BlockSpec((BR, f), lambda i: (i, 0)),
            pl.BlockSpec(W.shape, lambda i: (0, 0)),
            pl.BlockSpec((1, f), lambda i: (0, 0)),
        ],
        out_specs=pl.BlockSpec((BR, W.shape[1]), lambda i: (i, 0)),
        out_shape=jax.ShapeDtypeStruct((n, W.shape[1]), jnp.float32),
    )(x, W, b)


def _mlp_kernel(h, parts, Wa, ba, Wb, bb):
    n, f = h.shape

    def body(h_ref, p_ref, wa_ref, ba_ref, wb_ref, bb_ref, o_ref):
        m = h_ref[...] + p_ref[0] + p_ref[1]
        t = jnp.maximum(
            jnp.dot(m, wa_ref[...], preferred_element_type=jnp.float32)
            + ba_ref[...], 0.0)
        o_ref[...] = jnp.maximum(
            jnp.dot(t, wb_ref[...], preferred_element_type=jnp.float32)
            + bb_ref[...], 0.0)

    return pl.pallas_call(
        body,
        grid=(n // BR,),
        in_specs=[
            pl.BlockSpec((BR, f), lambda i: (i, 0)),
            pl.BlockSpec((NC, BR, f), lambda i: (0, i, 0)),
            pl.BlockSpec((f, f), lambda i: (0, 0)),
            pl.BlockSpec((1, f), lambda i: (0, 0)),
            pl.BlockSpec((f, f), lambda i: (0, 0)),
            pl.BlockSpec((1, f), lambda i: (0, 0)),
        ],
        out_specs=pl.BlockSpec((BR, f), lambda i: (i, 0)),
        out_shape=jax.ShapeDtypeStruct((n, f), jnp.float32),
    )(h, parts, Wa, ba, Wb, bb)


def _final_kernel(h, parts, Wa, ba, Wb, bb, W1, b1, W2p, b2p, nclass):
    n, f = h.shape
    nblk = n // BR

    def body(h_ref, p_ref, wa_ref, ba_ref, wb_ref, bb_ref,
             w1_ref, b1_ref, w2_ref, b2_ref, o_ref, acc):
        i = pl.program_id(0)

        @pl.when(i == 0)
        def _():
            acc[...] = jnp.zeros_like(acc)

        m = h_ref[...] + p_ref[0] + p_ref[1]
        t = jnp.maximum(
            jnp.dot(m, wa_ref[...], preferred_element_type=jnp.float32)
            + ba_ref[...], 0.0)
        t = jnp.maximum(
            jnp.dot(t, wb_ref[...], preferred_element_type=jnp.float32)
            + bb_ref[...], 0.0)
        t = jnp.maximum(
            jnp.dot(t, w1_ref[...], preferred_element_type=jnp.float32)
            + b1_ref[...], 0.0)
        acc[...] += jnp.sum(t, axis=0, keepdims=True)

        @pl.when(i == nblk - 1)
        def _():
            pooled = acc[...] * (1.0 / n)
            logits = (jnp.dot(pooled, w2_ref[...],
                              preferred_element_type=jnp.float32) + b2_ref[...])
            lane = lax.broadcasted_iota(jnp.int32, (1, f), 1)
            valid = lane < nclass
            mx = jnp.max(jnp.where(valid, logits, -jnp.inf), axis=1, keepdims=True)
            e = jnp.where(valid, jnp.exp(logits - mx), 0.0)
            o_ref[...] = e / jnp.sum(e, axis=1, keepdims=True)

    return pl.pallas_call(
        body,
        grid=(nblk,),
        in_specs=[
            pl.BlockSpec((BR, f), lambda i: (i, 0)),
            pl.BlockSpec((NC, BR, f), lambda i: (0, i, 0)),
            pl.BlockSpec((f, f), lambda i: (0, 0)),
            pl.BlockSpec((1, f), lambda i: (0, 0)),
            pl.BlockSpec((f, f), lambda i: (0, 0)),
            pl.BlockSpec((1, f), lambda i: (0, 0)),
            pl.BlockSpec((f, f), lambda i: (0, 0)),
            pl.BlockSpec((1, f), lambda i: (0, 0)),
            pl.BlockSpec((f, f), lambda i: (0, 0)),
            pl.BlockSpec((1, f), lambda i: (0, 0)),
        ],
        out_specs=pl.BlockSpec((1, f), lambda i: (0, 0)),
        out_shape=jax.ShapeDtypeStruct((1, f), jnp.float32),
        scratch_shapes=[pltpu.VMEM((1, f), jnp.float32)],
    )(h, parts, Wa, ba, Wb, bb, W1, b1, W2p, b2p)


def kernel(x, edge_index, batch, params):
    n, f = x.shape
    e = edge_index.shape[1]
    nclass = params['W2'].shape[1]
    bn_eps = 1e-5

    def fold(W, b, g, be):
        s = g / jnp.sqrt(1.0 + bn_eps)
        return W * s[None, :], (b * s + be)[None, :]

    W0 = params['W0']
    b0 = params['b0'][None, :]
    convs = []
    for p in params['convs']:
        WaF, baF = fold(p['Wa'], p['ba'], p['g1'], p['be1'])
        WbF, bbF = fold(p['Wb'], p['bb'], p['g2'], p['be2'])
        convs.append((WaF, baF, WbF, bbF))
    W1 = params['W1']
    b1 = params['b1'][None, :]
    W2p = jnp.zeros((f, f), jnp.float32).at[:, :nclass].set(params['W2'])
    b2p = jnp.zeros((1, f), jnp.float32).at[:, :nclass].set(params['b2'])

    chunks_total = _cdiv(e, CH)
    m0 = max(NBUF, int(chunks_total * F0 / NS) // NBUF * NBUF)
    m1 = _cdiv(_cdiv(max(e - NS * m0 * CH, 0), CH), NS * NBUF) * NS * NBUF // NS
    e0 = NS * m0 * CH
    epad = NS * (m0 + m1) * CH
    rows_pad = _cdiv(n + 1, NS * 8) * NS * 8
    pad = epad - e
    src = edge_index[0]
    dst = edge_index[1]
    src_p = jnp.concatenate([src, jnp.zeros((pad,), src.dtype)])
    dst_p = jnp.concatenate([dst, jnp.full((pad,), n, dst.dtype)])
    sd0 = jnp.stack([src_p[:e0].reshape(NS, m0, CH),
                     dst_p[:e0].reshape(NS, m0, CH)], axis=2)
    sd1 = jnp.stack([src_p[e0:].reshape(NS, m1, CH),
                     dst_p[e0:].reshape(NS, m1, CH)], axis=2)

    h = _entry_kernel(x, W0, b0)
    for i in range(len(convs)):
        parts = _sc_aggregate(h, sd0, sd1, rows_pad)
        WaF, baF, WbF, bbF = convs[i]
        if i + 1 < len(convs):
            h = _mlp_kernel(h, parts, WaF, baF, WbF, bbF)
        else:
            out = _final_kernel(h, parts, WaF, baF, WbF, bbF, W1, b1, W2p, b2p, nclass)
    return out[:, :nclass]


# speedup vs baseline: 1.6098x; 1.1511x over previous
import functools

import jax
import jax.numpy as jnp
from jax import lax
from jax.experimental import pallas as pl
from jax.experimental.pallas import tpu as pltpu
from jax.experimental.pallas import tpu_sc as plsc

NC = 2
NS = 16
NW = NC * NS
CH = 80
SECT = 40
NBUF = 2
F0 = 0.5
BR = 1000


def _cdiv(a, b):
    return (a + b - 1) // b


def _sc_aggregate(h, sd0, sd1, rows_pad):
    m_by_core = (sd0.shape[1], sd1.shape[1])
    F = h.shape[1]
    rows_per_tile = rows_pad // NS
    zfull = rows_per_tile // CH
    zrem = rows_per_tile - zfull * CH
    mesh = plsc.VectorSubcoreMesh(core_axis_name="c", subcore_axis_name="s")

    @functools.partial(
        pl.kernel,
        out_type=jax.ShapeDtypeStruct((NC, rows_pad, F), jnp.float32),
        mesh=mesh,
        scratch_types=[
            pltpu.VMEM((SECT, 2, CH), jnp.int32),
        ] + [pltpu.VMEM((CH, F), jnp.float32)] * NBUF
          + [pltpu.VMEM_SHARED((rows_pad, F), jnp.float32)]
          + [pltpu.SemaphoreType.DMA] * NBUF,
    )
    def agg_kernel(h_hbm, sd0_hbm, sd1_hbm, out_hbm, sdv, *rest):
        rows = rest[:NBUF]
        aggsh = rest[NBUF]
        sems = rest[NBUF + 1:]
        rowsa = rows[0]
        cid = lax.axis_index("c")
        sid = lax.axis_index("s")

        @pl.loop(0, CH)
        def _(r):
            @pl.loop(0, F, step=16)
            def _(c):
                rowsa[pl.ds(r, 1), pl.ds(c, 16)] = jnp.zeros((1, 16), jnp.float32)

        @pl.loop(0, zfull * CH, step=CH)
        def _(r):
            pltpu.sync_copy(rowsa, aggsh.at[pl.ds(sid * rows_per_tile + r, CH)])

        if zrem:
            pltpu.sync_copy(
                rowsa.at[pl.ds(0, zrem)],
                aggsh.at[pl.ds(sid * rows_per_tile + zfull * CH, zrem)],
            )

        plsc.subcore_barrier()

        def run_core(sd_hbm, m):
            for s0 in range(0, m, SECT):
                sw = min(SECT, m - s0)
                pltpu.sync_copy(sd_hbm.at[sid, pl.ds(s0, sw)], sdv.at[pl.ds(0, sw)])
                for k in range(NBUF - 1):
                    pltpu.async_copy(h_hbm.at[sdv.at[k, 0]], rows[k], sems[k])

                @pl.loop(0, sw, step=NBUF)
                def _(j):
                    for k in range(NBUF):
                        ki = (k + NBUF - 1) % NBUF
                        jn = jnp.where(j + k + NBUF - 1 < sw, j + k + NBUF - 1, 0)
                        pltpu.async_copy(h_hbm.at[sdv.at[jn, 0]], rows[ki], sems[ki])
                        pltpu.make_async_copy(h_hbm.at[sdv.at[0, 0]], rows[k], sems[k]).wait()
                        pltpu.sync_copy(rows[k], aggsh.at[sdv.at[j + k, 1]], add=True)

                for k in range(NBUF - 1):
                    pltpu.make_async_copy(h_hbm.at[sdv.at[0, 0]], rows[k], sems[k]).wait()

        @pl.when(cid == 0)
        def _():
            run_core(sd0_hbm, m_by_core[0])

        @pl.when(cid == 1)
        def _():
            run_core(sd1_hbm, m_by_core[1])

        plsc.subcore_barrier()
        pltpu.sync_copy(
            aggsh.at[pl.ds(sid * rows_per_tile, rows_per_tile)],
            out_hbm.at[cid, pl.ds(sid * rows_per_tile, rows_per_tile)],
        )

    return agg_kernel(h, sd0, sd1)


def _entry_kernel(x, W, b):
    n, f = x.shape

    def body(x_ref, w_ref, b_ref, o_ref):
        o_ref[...] = jnp.maximum(
            jnp.dot(x_ref[...], w_ref[...], preferred_element_type=jnp.float32)
            + b_ref[...], 0.0)

    return pl.pallas_call(
        body,
        grid=(n // BR,),
        in_specs=[
            pl.BlockSpec((BR, f), lambda i: (i, 0)),
            pl.BlockSpec(W.shape, lambda i: (0, 0)),
            pl.BlockSpec((1, f), lambda i: (0, 0)),
        ],
        out_specs=pl.BlockSpec((BR, W.shape[1]), lambda i: (i, 0)),
        out_shape=jax.ShapeDtypeStruct((n, W.shape[1]), jnp.float32),
    )(x, W, b)


def _mlp_kernel(h, parts, Wa, ba, Wb, bb):
    n, f = h.shape

    def body(h_ref, p_ref, wa_ref, ba_ref, wb_ref, bb_ref, o_ref):
        m = h_ref[...] + p_ref[0] + p_ref[1]
        t = jnp.maximum(
            jnp.dot(m, wa_ref[...], preferred_element_type=jnp.float32)
            + ba_ref[...], 0.0)
        o_ref[...] = jnp.maximum(
            jnp.dot(t, wb_ref[...], preferred_element_type=jnp.float32)
            + bb_ref[...], 0.0)

    return pl.pallas_call(
        body,
        grid=(n // BR,),
        in_specs=[
            pl.BlockSpec((BR, f), lambda i: (i, 0)),
            pl.BlockSpec((NC, BR, f), lambda i: (0, i, 0)),
            pl.BlockSpec((f, f), lambda i: (0, 0)),
            pl.BlockSpec((1, f), lambda i: (0, 0)),
            pl.BlockSpec((f, f), lambda i: (0, 0)),
            pl.BlockSpec((1, f), lambda i: (0, 0)),
        ],
        out_specs=pl.BlockSpec((BR, f), lambda i: (i, 0)),
        out_shape=jax.ShapeDtypeStruct((n, f), jnp.float32),
    )(h, parts, Wa, ba, Wb, bb)


def _final_kernel(h, parts, Wa, ba, Wb, bb, W1, b1, W2p, b2p, nclass):
    n, f = h.shape
    nblk = n // BR

    def body(h_ref, p_ref, wa_ref, ba_ref, wb_ref, bb_ref,
             w1_ref, b1_ref, w2_ref, b2_ref, o_ref, acc):
        i = pl.program_id(0)

        @pl.when(i == 0)
        def _():
            acc[...] = jnp.zeros_like(acc)

        m = h_ref[...] + p_ref[0] + p_ref[1]
        t = jnp.maximum(
            jnp.dot(m, wa_ref[...], preferred_element_type=jnp.float32)
            + ba_ref[...], 0.0)
        t = jnp.maximum(
            jnp.dot(t, wb_ref[...], preferred_element_type=jnp.float32)
            + bb_ref[...], 0.0)
        t = jnp.maximum(
            jnp.dot(t, w1_ref[...], preferred_element_type=jnp.float32)
            + b1_ref[...], 0.0)
        acc[...] += jnp.sum(t, axis=0, keepdims=True)

        @pl.when(i == nblk - 1)
        def _():
            pooled = acc[...] * (1.0 / n)
            logits = (jnp.dot(pooled, w2_ref[...],
                              preferred_element_type=jnp.float32) + b2_ref[...])
            lane = lax.broadcasted_iota(jnp.int32, (1, f), 1)
            valid = lane < nclass
            mx = jnp.max(jnp.where(valid, logits, -jnp.inf), axis=1, keepdims=True)
            e = jnp.where(valid, jnp.exp(logits - mx), 0.0)
            o_ref[...] = e / jnp.sum(e, axis=1, keepdims=True)

    return pl.pallas_call(
        body,
        grid=(nblk,),
        in_specs=[
            pl.BlockSpec((BR, f), lambda i: (i, 0)),
            pl.BlockSpec((NC, BR, f), lambda i: (0, i, 0)),
            pl.BlockSpec((f, f), lambda i: (0, 0)),
            pl.BlockSpec((1, f), lambda i: (0, 0)),
            pl.BlockSpec((f, f), lambda i: (0, 0)),
            pl.BlockSpec((1, f), lambda i: (0, 0)),
            pl.BlockSpec((f, f), lambda i: (0, 0)),
            pl.BlockSpec((1, f), lambda i: (0, 0)),
            pl.BlockSpec((f, f), lambda i: (0, 0)),
            pl.BlockSpec((1, f), lambda i: (0, 0)),
        ],
        out_specs=pl.BlockSpec((1, f), lambda i: (0, 0)),
        out_shape=jax.ShapeDtypeStruct((1, f), jnp.float32),
        scratch_shapes=[pltpu.VMEM((1, f), jnp.float32)],
    )(h, parts, Wa, ba, Wb, bb, W1, b1, W2p, b2p)


def kernel(x, edge_index, batch, params):
    n, f = x.shape
    e = edge_index.shape[1]
    nclass = params['W2'].shape[1]
    bn_eps = 1e-5

    def fold(W, b, g, be):
        s = g / jnp.sqrt(1.0 + bn_eps)
        return W * s[None, :], (b * s + be)[None, :]

    W0 = params['W0']
    b0 = params['b0'][None, :]
    convs = []
    for p in params['convs']:
        WaF, baF = fold(p['Wa'], p['ba'], p['g1'], p['be1'])
        WbF, bbF = fold(p['Wb'], p['bb'], p['g2'], p['be2'])
        convs.append((WaF, baF, WbF, bbF))
    W1 = params['W1']
    b1 = params['b1'][None, :]
    W2p = jnp.zeros((f, f), jnp.float32).at[:, :nclass].set(params['W2'])
    b2p = jnp.zeros((1, f), jnp.float32).at[:, :nclass].set(params['b2'])

    chunks_total = _cdiv(e, CH)
    m0 = max(NBUF, int(chunks_total * F0 / NS) // NBUF * NBUF)
    m1 = _cdiv(_cdiv(max(e - NS * m0 * CH, 0), CH), NS * NBUF) * NS * NBUF // NS
    e0 = NS * m0 * CH
    epad = NS * (m0 + m1) * CH
    rows_pad = _cdiv(n + 1, NS * 8) * NS * 8
    pad = epad - e
    src = edge_index[0]
    dst = edge_index[1]
    src_p = jnp.concatenate([src, jnp.zeros((pad,), src.dtype)])
    dst_p = jnp.concatenate([dst, jnp.full((pad,), n, dst.dtype)])
    sd0 = jnp.stack([src_p[:e0].reshape(NS, m0, CH),
                     dst_p[:e0].reshape(NS, m0, CH)], axis=2)
    sd1 = jnp.stack([src_p[e0:].reshape(NS, m1, CH),
                     dst_p[e0:].reshape(NS, m1, CH)], axis=2)

    h = _entry_kernel(x, W0, b0)
    for i in range(len(convs)):
        parts = _sc_aggregate(h, sd0, sd1, rows_pad)
        WaF, baF, WbF, bbF = convs[i]
        if i + 1 < len(convs):
            h = _mlp_kernel(h, parts, WaF, baF, WbF, bbF)
        else:
            out = _final_kernel(h, parts, WaF, baF, WbF, bbF, W1, b1, W2p, b2p, nclass)
    return out[:, :nclass]
